# Initial kernel scaffold; baseline (speedup 1.0000x reference)
import jax
import jax.numpy as jnp
from jax.experimental import pallas as pl


def kernel(topic, score, time, h, vs, hs, emb, W_ih, W_hh, b_ih, b_hh, W_score, b_score):
    raise NotImplementedError("write your pallas kernel here")


# speedup vs baseline: 1.1690x; 1.1690x over previous
import functools

import jax
import jax.numpy as jnp
from jax import lax
from jax.experimental import pallas as pl
from jax.experimental.pallas import tpu as pltpu
from jax.experimental.pallas import tpu_sc as plsc

TS = 128
HS = 128
K = 64
T = 32768
L = 64

BLK = 2048
NB = T // BLK
MID = NB
NSTEPS = 2 * NB + 1

_HI = jax.lax.Precision.HIGHEST


def _sc_topic_mean(topic, emb):
    mesh = plsc.VectorSubcoreMesh(core_axis_name="c", subcore_axis_name="s")

    @functools.partial(
        pl.kernel,
        out_type=jax.ShapeDtypeStruct((TS,), jnp.float32),
        mesh=mesh,
        scratch_types=[
            pltpu.VMEM((L,), jnp.int32),
            pltpu.VMEM((L, TS), jnp.float32),
            pltpu.VMEM((16,), jnp.float32),
            pltpu.SemaphoreType.DMA,
        ],
    )
    def k(topic_hbm, emb_hbm, out_hbm, idx_v, rows_v, acc_v, sem):
        cid = lax.axis_index("c")
        sid = lax.axis_index("s")

        @pl.when(jnp.logical_and(cid == 0, sid < TS // 16))
        def _():
            pltpu.sync_copy(topic_hbm, idx_v)
            pltpu.async_copy(emb_hbm.at[idx_v], rows_v, sem).wait()
            chunk = pl.ds(sid * 16, 16)

            def body(r, acc):
                return acc + rows_v[r, chunk]

            acc = lax.fori_loop(0, L, body, jnp.zeros((16,), jnp.float32))
            acc_v[...] = acc * (1.0 / L)
            pltpu.sync_copy(acc_v, out_hbm.at[chunk])

    return k(topic, emb)


def _sortable(x):
    u = lax.bitcast_convert_type(x, jnp.int32)
    return jnp.where(u >= 0, u, u ^ jnp.int32(0x7FFFFFFF))


def _tc_body(v_ref, s_ref, h_ref, vs_ref, hs_ref, wihT_ref, whhT_ref,
             bih_ref, bhh_ref, wsc_ref, bsc_ref,
             sc_out, hnew_out, vsn_out, hsn_out,
             alpha_ref, acc_ref, stat_ref):
    i = pl.program_id(0)

    @pl.when(i < NB)
    def _vs_phase():
        blk = vs_ref[...]
        vsn_out[...] = blk
        a_row = lax.dot_general(v_ref[...], blk, (((1,), (1,)), ((), ())),
                                preferred_element_type=jnp.float32,
                                precision=_HI)
        alpha_ref[pl.ds(i, 1), :] = a_row

    @pl.when(i == MID)
    def _mid_phase():
        v = v_ref[...]
        vsn_out[0:1, :] = v

        s = s_ref[0, 0]
        ge = (s >= 0.5).astype(jnp.float32)
        xa = v * ge
        xb = v * (1.0 - ge)
        gi = (lax.dot_general(xa, wihT_ref[0:TS, :], (((1,), (0,)), ((), ())),
                              preferred_element_type=jnp.float32, precision=_HI)
              + lax.dot_general(xb, wihT_ref[TS:2 * TS, :],
                                (((1,), (0,)), ((), ())),
                                preferred_element_type=jnp.float32,
                                precision=_HI)
              + s * wihT_ref[2 * TS:2 * TS + 1, :]
              + bih_ref[...])
        hrow = h_ref[...]
        gh = (lax.dot_general(hrow, whhT_ref[...], (((1,), (0,)), ((), ())),
                              preferred_element_type=jnp.float32, precision=_HI)
              + bhh_ref[...])
        r = jax.nn.sigmoid(gi[:, 0:HS] + gh[:, 0:HS])
        z = jax.nn.sigmoid(gi[:, HS:2 * HS] + gh[:, HS:2 * HS])
        n = jnp.tanh(gi[:, 2 * HS:3 * HS] + r * gh[:, 2 * HS:3 * HS])
        h_new = (1.0 - z) * n + z * hrow
        hnew_out[...] = h_new
        hsn_out[0:1, :] = h_new

        al = alpha_ref[...]
        m = jnp.max(al)
        key = _sortable(al)

        def bis(_, lohi):
            lo, hi = lohi
            mid = (lo >> 1) + (hi >> 1) + (lo & hi & 1)
            cnt = jnp.sum((key >= mid).astype(jnp.int32))
            return jnp.where(cnt >= K, mid, lo), jnp.where(cnt >= K, hi, mid - 1)

        lo, hi = lax.fori_loop(0, 34, bis, (jnp.min(key), jnp.max(key)))
        w = jnp.where(key >= lo, jnp.exp(al - m), 0.0)
        alpha_ref[...] = w
        stat_ref[0] = jnp.sum(w)
        acc_ref[...] = jnp.zeros((1, HS), jnp.float32)

    @pl.when(i > MID)
    def _hs_phase():
        j = i - (MID + 1)
        blk = hs_ref[...]
        hsn_out[...] = blk
        w_row = alpha_ref[pl.ds(j, 1), :]
        acc_ref[...] += lax.dot_general(w_row, blk, (((1,), (0,)), ((), ())),
                                        preferred_element_type=jnp.float32,
                                        precision=_HI)

    @pl.when(i == NSTEPS - 1)
    def _fin():
        attn = acc_ref[...] / stat_ref[0]
        sc = (jnp.sum(v_ref[...] * wsc_ref[:, 0:TS])
              + jnp.sum(attn * wsc_ref[:, TS:TS + HS])
              + jnp.sum(h_ref[...] * wsc_ref[:, TS + HS:TS + 2 * HS])
              + wsc_ref[0, TS + 2 * HS] * float(K)
              + bsc_ref[0, 0])
        sc_out[...] = sc.reshape(1, 1)


def _tc_main(v_row, score, h_row, vs, hs2, wihT, whhT, bih, bhh, wsc, bsc):
    whole = lambda shape: pl.BlockSpec(shape, lambda i: tuple(0 for _ in shape))
    return pl.pallas_call(
        _tc_body,
        grid=(NSTEPS,),
        in_specs=[
            whole((1, TS)),
            whole((1, 1)),
            whole((1, HS)),
            pl.BlockSpec((BLK, TS), lambda i: (jnp.minimum(i, NB - 1), 0)),
            pl.BlockSpec((BLK, HS),
                         lambda i: (jnp.clip(i - (MID + 1), 0, NB - 1), 0)),
            whole((2 * TS + 1, 3 * HS)),
            whole((HS, 3 * HS)),
            whole((1, 3 * HS)),
            whole((1, 3 * HS)),
            whole((1, TS + 2 * HS + 1)),
            whole((1, 1)),
        ],
        out_specs=[
            whole((1, 1)),
            whole((1, HS)),
            pl.BlockSpec((BLK, TS), lambda i: (jnp.minimum(i, NB), 0)),
            pl.BlockSpec((BLK, HS),
                         lambda i: (jnp.where(i > MID, i - (MID + 1), NB), 0)),
        ],
        out_shape=[
            jax.ShapeDtypeStruct((1, 1), jnp.float32),
            jax.ShapeDtypeStruct((1, HS), jnp.float32),
            jax.ShapeDtypeStruct((T + 1, TS), jnp.float32),
            jax.ShapeDtypeStruct((T + 1, HS), jnp.float32),
        ],
        scratch_shapes=[
            pltpu.VMEM((NB, BLK), jnp.float32),
            pltpu.VMEM((1, HS), jnp.float32),
            pltpu.SMEM((2,), jnp.float32),
        ],
    )(v_row, score, h_row, vs, hs2, wihT, whhT, bih, bhh, wsc, bsc)


def kernel(topic, score, time, h, vs, hs, emb, W_ih, W_hh, b_ih, b_hh,
           W_score, b_score):
    del time
    v = _sc_topic_mean(topic.astype(jnp.int32), emb)
    sc, h_new, vs_new, hs_new = _tc_main(
        v.reshape(1, TS), score.reshape(1, 1), h.reshape(1, HS),
        vs, hs.reshape(T, HS),
        W_ih.T, W_hh.T, b_ih.reshape(1, -1), b_hh.reshape(1, -1),
        W_score, b_score.reshape(1, 1))
    return (sc, h_new.reshape(1, 1, HS), vs_new,
            hs_new.reshape(T + 1, 1, HS))


# speedup vs baseline: 1.2810x; 1.0958x over previous
import functools

import jax
import jax.numpy as jnp
from jax import lax
from jax.experimental import pallas as pl
from jax.experimental.pallas import tpu as pltpu
from jax.experimental.pallas import tpu_sc as plsc

TS = 128
HS = 128
K = 64
T = 32768
L = 64

BLK = 2048
NB = T // BLK
MID = NB
NSTEPS = 2 * NB + 1

_HI = jax.lax.Precision.HIGHEST


def _sc_topic_mean(topic, emb):
    mesh = plsc.VectorSubcoreMesh(core_axis_name="c", subcore_axis_name="s")

    @functools.partial(
        pl.kernel,
        out_type=jax.ShapeDtypeStruct((TS,), jnp.float32),
        mesh=mesh,
        scratch_types=[
            pltpu.VMEM((L,), jnp.int32),
            pltpu.VMEM((L, TS), jnp.float32),
            pltpu.VMEM((16,), jnp.float32),
            pltpu.SemaphoreType.DMA,
        ],
    )
    def k(topic_hbm, emb_hbm, out_hbm, idx_v, rows_v, acc_v, sem):
        cid = lax.axis_index("c")
        sid = lax.axis_index("s")

        @pl.when(jnp.logical_and(cid == 0, sid < TS // 16))
        def _():
            pltpu.sync_copy(topic_hbm, idx_v)
            pltpu.async_copy(emb_hbm.at[idx_v], rows_v, sem).wait()
            chunk = pl.ds(sid * 16, 16)

            def body(r, acc):
                return acc + rows_v[r, chunk]

            acc = lax.fori_loop(0, L, body, jnp.zeros((16,), jnp.float32))
            acc_v[...] = acc * (1.0 / L)
            pltpu.sync_copy(acc_v, out_hbm.at[chunk])

    return k(topic, emb)


def _sortable(x):
    u = lax.bitcast_convert_type(x, jnp.int32)
    return jnp.where(u >= 0, u, u ^ jnp.int32(0x7FFFFFFF))


def _tc_body(v_ref, s_ref, h_ref, vs_ref, hs_ref, wihT_ref, whhT_ref,
             bih_ref, bhh_ref, wsc_ref, bsc_ref,
             sc_out, hnew_out, vsn_out, hsn_out,
             alpha_ref, acc_ref, stat_ref):
    i = pl.program_id(0)

    @pl.when(i < NB)
    def _vs_phase():
        blk = vs_ref[...]
        vsn_out[...] = blk
        a_row = lax.dot_general(v_ref[...], blk, (((1,), (1,)), ((), ())),
                                preferred_element_type=jnp.float32,
                                precision=_HI)
        alpha_ref[pl.ds(i, 1), :] = a_row

    @pl.when(i == MID)
    def _mid_phase():
        v = v_ref[...]
        vsn_out[0:1, :] = v

        s = s_ref[0, 0]
        ge = (s >= 0.5).astype(jnp.float32)
        xa = v * ge
        xb = v * (1.0 - ge)
        gi = (lax.dot_general(xa, wihT_ref[0:TS, :], (((1,), (0,)), ((), ())),
                              preferred_element_type=jnp.float32, precision=_HI)
              + lax.dot_general(xb, wihT_ref[TS:2 * TS, :],
                                (((1,), (0,)), ((), ())),
                                preferred_element_type=jnp.float32,
                                precision=_HI)
              + s * wihT_ref[2 * TS:2 * TS + 1, :]
              + bih_ref[...])
        hrow = h_ref[...]
        gh = (lax.dot_general(hrow, whhT_ref[...], (((1,), (0,)), ((), ())),
                              preferred_element_type=jnp.float32, precision=_HI)
              + bhh_ref[...])
        r = jax.nn.sigmoid(gi[:, 0:HS] + gh[:, 0:HS])
        z = jax.nn.sigmoid(gi[:, HS:2 * HS] + gh[:, HS:2 * HS])
        n = jnp.tanh(gi[:, 2 * HS:3 * HS] + r * gh[:, 2 * HS:3 * HS])
        h_new = (1.0 - z) * n + z * hrow
        hnew_out[...] = h_new
        hsn_out[0:1, 0, :] = h_new

        al = alpha_ref[...]
        m = jnp.max(al)
        key = _sortable(al)

        def bis(_, lohi):
            lo, hi = lohi
            mid = (lo >> 1) + (hi >> 1) + (lo & hi & 1)
            cnt = jnp.sum((key >= mid).astype(jnp.int32))
            return jnp.where(cnt >= K, mid, lo), jnp.where(cnt >= K, hi, mid - 1)

        lo, hi = lax.fori_loop(0, 34, bis, (jnp.min(key), jnp.max(key)))
        w = jnp.where(key >= lo, jnp.exp(al - m), 0.0)
        alpha_ref[...] = w
        stat_ref[0] = jnp.sum(w)
        acc_ref[...] = jnp.zeros((1, HS), jnp.float32)

    @pl.when(i > MID)
    def _hs_phase():
        j = i - (MID + 1)
        blk = hs_ref[:, 0, :]
        hsn_out[:, 0, :] = blk
        w_row = alpha_ref[pl.ds(j, 1), :]
        acc_ref[...] += lax.dot_general(w_row, blk, (((1,), (0,)), ((), ())),
                                        preferred_element_type=jnp.float32,
                                        precision=_HI)

    @pl.when(i == NSTEPS - 1)
    def _fin():
        attn = acc_ref[...] / stat_ref[0]
        sc = (jnp.sum(v_ref[...] * wsc_ref[:, 0:TS])
              + jnp.sum(attn * wsc_ref[:, TS:TS + HS])
              + jnp.sum(h_ref[...] * wsc_ref[:, TS + HS:TS + 2 * HS])
              + wsc_ref[0, TS + 2 * HS] * float(K)
              + bsc_ref[0, 0])
        sc_out[...] = sc.reshape(1, 1)


def _tc_main(v_row, score, h_row, vs, hs2, wihT, whhT, bih, bhh, wsc, bsc):
    whole = lambda shape: pl.BlockSpec(shape, lambda i: tuple(0 for _ in shape))
    return pl.pallas_call(
        _tc_body,
        grid=(NSTEPS,),
        in_specs=[
            whole((1, TS)),
            whole((1, 1)),
            whole((1, HS)),
            pl.BlockSpec((BLK, TS), lambda i: (jnp.minimum(i, NB - 1), 0)),
            pl.BlockSpec((BLK, 1, HS),
                         lambda i: (jnp.clip(i - (MID + 1), 0, NB - 1), 0, 0)),
            whole((2 * TS + 1, 3 * HS)),
            whole((HS, 3 * HS)),
            whole((1, 3 * HS)),
            whole((1, 3 * HS)),
            whole((1, TS + 2 * HS + 1)),
            whole((1, 1)),
        ],
        out_specs=[
            whole((1, 1)),
            whole((1, HS)),
            pl.BlockSpec((BLK, TS), lambda i: (jnp.minimum(i, NB), 0)),
            pl.BlockSpec((BLK, 1, HS),
                         lambda i: (jnp.where(i > MID, i - (MID + 1), NB), 0, 0)),
        ],
        out_shape=[
            jax.ShapeDtypeStruct((1, 1), jnp.float32),
            jax.ShapeDtypeStruct((1, HS), jnp.float32),
            jax.ShapeDtypeStruct((T + 1, TS), jnp.float32),
            jax.ShapeDtypeStruct((T + 1, 1, HS), jnp.float32),
        ],
        scratch_shapes=[
            pltpu.VMEM((NB, BLK), jnp.float32),
            pltpu.VMEM((1, HS), jnp.float32),
            pltpu.SMEM((2,), jnp.float32),
        ],
    )(v_row, score, h_row, vs, hs2, wihT, whhT, bih, bhh, wsc, bsc)


def kernel(topic, score, time, h, vs, hs, emb, W_ih, W_hh, b_ih, b_hh,
           W_score, b_score):
    del time
    v = _sc_topic_mean(topic.astype(jnp.int32), emb)
    sc, h_new, vs_new, hs_new = _tc_main(
        v.reshape(1, TS), score.reshape(1, 1), h.reshape(1, HS),
        vs, hs,
        W_ih.T, W_hh.T, b_ih.reshape(1, -1), b_hh.reshape(1, -1),
        W_score, b_score.reshape(1, 1))
    return sc, h_new.reshape(1, 1, HS), vs_new, hs_new


# speedup vs baseline: 1.6208x; 1.2653x over previous
import functools

import jax
import jax.numpy as jnp
from jax import lax
from jax.experimental import pallas as pl
from jax.experimental.pallas import tpu as pltpu
from jax.experimental.pallas import tpu_sc as plsc

TS = 128
HS = 128
K = 64
T = 32768
L = 64

BLK = 2048
NB = T // BLK
MID = NB
NSTEPS = 2 * NB + 1

_HI = jax.lax.Precision.HIGHEST


def _sc_topic_mean(topic, emb):
    mesh = plsc.VectorSubcoreMesh(core_axis_name="c", subcore_axis_name="s")

    @functools.partial(
        pl.kernel,
        out_type=jax.ShapeDtypeStruct((TS,), jnp.float32),
        mesh=mesh,
        scratch_types=[
            pltpu.VMEM((L,), jnp.int32),
            pltpu.VMEM((L, TS), jnp.float32),
            pltpu.VMEM((16,), jnp.float32),
            pltpu.SemaphoreType.DMA,
        ],
    )
    def k(topic_hbm, emb_hbm, out_hbm, idx_v, rows_v, acc_v, sem):
        cid = lax.axis_index("c")
        sid = lax.axis_index("s")

        @pl.when(jnp.logical_and(cid == 0, sid < TS // 16))
        def _():
            pltpu.sync_copy(topic_hbm, idx_v)
            pltpu.async_copy(emb_hbm.at[idx_v], rows_v, sem).wait()
            chunk = pl.ds(sid * 16, 16)

            def body(r, acc):
                return acc + rows_v[r, chunk]

            acc = lax.fori_loop(0, L, body, jnp.zeros((16,), jnp.float32))
            acc_v[...] = acc * (1.0 / L)
            pltpu.sync_copy(acc_v, out_hbm.at[chunk])

    return k(topic, emb)


def _sortable(x):
    u = lax.bitcast_convert_type(x, jnp.int32)
    return jnp.where(u >= 0, u, u ^ jnp.int32(0x7FFFFFFF))


def _tc_body(v_ref, s_ref, h_ref, vs_ref, hs_ref, wihT_ref, whhT_ref,
             bih_ref, bhh_ref, wsc_ref, bsc_ref,
             sc_out, hnew_out, vsn_out, hsn_out,
             alpha_ref, acc_ref, hrow_ref, stat_ref, hsbuf,
             in_sems, out_sems, last_sem):
    i = pl.program_id(0)

    @pl.when(i < NB)
    def _vs_phase():
        pltpu.make_async_copy(hs_ref.at[pl.ds(i * BLK, BLK), 0],
                              hsbuf.at[i], in_sems.at[i]).start()
        blk = vs_ref[...]
        vsn_out[...] = blk
        a_row = lax.dot_general(v_ref[...], blk, (((1,), (1,)), ((), ())),
                                preferred_element_type=jnp.float32,
                                precision=_HI)
        alpha_ref[pl.ds(i, 1), :] = a_row

    @pl.when(i == MID)
    def _mid_phase():
        v = v_ref[...]
        vsn_out[0:1, :] = v

        s = s_ref[0, 0]
        ge = (s >= 0.5).astype(jnp.float32)
        xa = v * ge
        xb = v * (1.0 - ge)
        gi = (lax.dot_general(xa, wihT_ref[0:TS, :], (((1,), (0,)), ((), ())),
                              preferred_element_type=jnp.float32, precision=_HI)
              + lax.dot_general(xb, wihT_ref[TS:2 * TS, :],
                                (((1,), (0,)), ((), ())),
                                preferred_element_type=jnp.float32,
                                precision=_HI)
              + s * wihT_ref[2 * TS:2 * TS + 1, :]
              + bih_ref[...])
        hrow = h_ref[...]
        gh = (lax.dot_general(hrow, whhT_ref[...], (((1,), (0,)), ((), ())),
                              preferred_element_type=jnp.float32, precision=_HI)
              + bhh_ref[...])
        r = jax.nn.sigmoid(gi[:, 0:HS] + gh[:, 0:HS])
        z = jax.nn.sigmoid(gi[:, HS:2 * HS] + gh[:, HS:2 * HS])
        n = jnp.tanh(gi[:, 2 * HS:3 * HS] + r * gh[:, 2 * HS:3 * HS])
        h_new = (1.0 - z) * n + z * hrow
        hnew_out[...] = h_new
        hrow_ref[...] = h_new
        pltpu.make_async_copy(hrow_ref, hsn_out.at[pl.ds(T, 1), 0],
                              last_sem).start()

        al = alpha_ref[...]
        m = jnp.max(al)
        key = _sortable(al)

        def bis(_, lohi):
            lo, hi = lohi
            mid = (lo >> 1) + (hi >> 1) + (lo & hi & 1)
            cnt = jnp.sum((key >= mid).astype(jnp.int32))
            return jnp.where(cnt >= K, mid, lo), jnp.where(cnt >= K, hi, mid - 1)

        lo, hi = lax.fori_loop(0, 34, bis, (jnp.min(key), jnp.max(key)))
        w = jnp.where(key >= lo, jnp.exp(al - m), 0.0)
        alpha_ref[...] = w
        stat_ref[0] = jnp.sum(w)
        acc_ref[...] = jnp.zeros((1, HS), jnp.float32)

    @pl.when(i > MID)
    def _hs_phase():
        j = i - (MID + 1)
        pltpu.make_async_copy(hs_ref.at[pl.ds(j * BLK, BLK), 0],
                              hsbuf.at[j], in_sems.at[j]).wait()
        blk = hsbuf[j]
        w_row = alpha_ref[pl.ds(j, 1), :]
        acc_ref[...] += lax.dot_general(w_row, blk, (((1,), (0,)), ((), ())),
                                        preferred_element_type=jnp.float32,
                                        precision=_HI)
        pltpu.make_async_copy(hsbuf.at[j], hsn_out.at[pl.ds(j * BLK, BLK), 0],
                              out_sems.at[j]).start()

    @pl.when(i == NSTEPS - 1)
    def _fin():
        attn = acc_ref[...] / stat_ref[0]
        sc = (jnp.sum(v_ref[...] * wsc_ref[:, 0:TS])
              + jnp.sum(attn * wsc_ref[:, TS:TS + HS])
              + jnp.sum(h_ref[...] * wsc_ref[:, TS + HS:TS + 2 * HS])
              + wsc_ref[0, TS + 2 * HS] * float(K)
              + bsc_ref[0, 0])
        sc_out[...] = sc.reshape(1, 1)
        for j in range(NB):
            pltpu.make_async_copy(hsbuf.at[j],
                                  hsn_out.at[pl.ds(j * BLK, BLK), 0],
                                  out_sems.at[j]).wait()
        pltpu.make_async_copy(hrow_ref, hsn_out.at[pl.ds(T, 1), 0],
                              last_sem).wait()


def _tc_main(v_row, score, h_row, vs, hs2, wihT, whhT, bih, bhh, wsc, bsc):
    whole = lambda shape: pl.BlockSpec(shape, lambda i: tuple(0 for _ in shape))
    return pl.pallas_call(
        _tc_body,
        grid=(NSTEPS,),
        in_specs=[
            whole((1, TS)),
            whole((1, 1)),
            whole((1, HS)),
            pl.BlockSpec((BLK, TS), lambda i: (jnp.minimum(i, NB - 1), 0)),
            pl.BlockSpec(memory_space=pl.ANY),
            whole((2 * TS + 1, 3 * HS)),
            whole((HS, 3 * HS)),
            whole((1, 3 * HS)),
            whole((1, 3 * HS)),
            whole((1, TS + 2 * HS + 1)),
            whole((1, 1)),
        ],
        out_specs=[
            whole((1, 1)),
            whole((1, HS)),
            pl.BlockSpec((BLK, TS), lambda i: (jnp.minimum(i, NB), 0)),
            pl.BlockSpec(memory_space=pl.ANY),
        ],
        out_shape=[
            jax.ShapeDtypeStruct((1, 1), jnp.float32),
            jax.ShapeDtypeStruct((1, HS), jnp.float32),
            jax.ShapeDtypeStruct((T + 1, TS), jnp.float32),
            jax.ShapeDtypeStruct((T + 1, 1, HS), jnp.float32),
        ],
        scratch_shapes=[
            pltpu.VMEM((NB, BLK), jnp.float32),
            pltpu.VMEM((1, HS), jnp.float32),
            pltpu.VMEM((1, HS), jnp.float32),
            pltpu.SMEM((2,), jnp.float32),
            pltpu.VMEM((NB, BLK, HS), jnp.float32),
            pltpu.SemaphoreType.DMA((NB,)),
            pltpu.SemaphoreType.DMA((NB,)),
            pltpu.SemaphoreType.DMA,
        ],
    )(v_row, score, h_row, vs, hs2, wihT, whhT, bih, bhh, wsc, bsc)


def kernel(topic, score, time, h, vs, hs, emb, W_ih, W_hh, b_ih, b_hh,
           W_score, b_score):
    del time
    v = _sc_topic_mean(topic.astype(jnp.int32), emb)
    sc, h_new, vs_new, hs_new = _tc_main(
        v.reshape(1, TS), score.reshape(1, 1), h.reshape(1, HS),
        vs, hs,
        W_ih.T, W_hh.T, b_ih.reshape(1, -1), b_hh.reshape(1, -1),
        W_score, b_score.reshape(1, 1))
    return sc, h_new.reshape(1, 1, HS), vs_new, hs_new


# speedup vs baseline: 1.7839x; 1.1006x over previous
import functools

import jax
import jax.numpy as jnp
from jax import lax
from jax.experimental import pallas as pl
from jax.experimental.pallas import tpu as pltpu
from jax.experimental.pallas import tpu_sc as plsc

TS = 128
HS = 128
K = 64
T = 32768
L = 64

BLK = 4096
NB = T // BLK
MID = NB
NSTEPS = 2 * NB + 1

_HI = jax.lax.Precision.HIGHEST


def _sc_topic_mean(topic, emb):
    mesh = plsc.VectorSubcoreMesh(core_axis_name="c", subcore_axis_name="s")

    @functools.partial(
        pl.kernel,
        out_type=jax.ShapeDtypeStruct((TS,), jnp.float32),
        mesh=mesh,
        scratch_types=[
            pltpu.VMEM((L,), jnp.int32),
            pltpu.VMEM((L, TS), jnp.float32),
            pltpu.VMEM((16,), jnp.float32),
            pltpu.SemaphoreType.DMA,
        ],
    )
    def k(topic_hbm, emb_hbm, out_hbm, idx_v, rows_v, acc_v, sem):
        cid = lax.axis_index("c")
        sid = lax.axis_index("s")

        @pl.when(jnp.logical_and(cid == 0, sid < TS // 16))
        def _():
            pltpu.sync_copy(topic_hbm, idx_v)
            pltpu.async_copy(emb_hbm.at[idx_v], rows_v, sem).wait()
            chunk = pl.ds(sid * 16, 16)

            def body(r, acc):
                return acc + rows_v[r, chunk]

            acc = lax.fori_loop(0, L, body, jnp.zeros((16,), jnp.float32))
            acc_v[...] = acc * (1.0 / L)
            pltpu.sync_copy(acc_v, out_hbm.at[chunk])

    return k(topic, emb)


def _sortable(x):
    u = lax.bitcast_convert_type(x, jnp.int32)
    return jnp.where(u >= 0, u, u ^ jnp.int32(0x7FFFFFFF))


def _tc_body(v_ref, s_ref, h_ref, vs_ref, hs_ref, wihT_ref, whhT_ref,
             bih_ref, bhh_ref, wsc_ref, bsc_ref,
             sc_out, hnew_out, vsn_out, hsn_out,
             alpha_ref, acc_ref, hrow_ref, stat_ref, hsbuf,
             in_sems, out_sems, last_sem):
    i = pl.program_id(0)

    @pl.when(i < NB)
    def _vs_phase():
        pltpu.make_async_copy(hs_ref.at[pl.ds(i * BLK, BLK), 0],
                              hsbuf.at[i], in_sems.at[i]).start()
        blk = vs_ref[...]
        vsn_out[...] = blk
        a_row = lax.dot_general(v_ref[...], blk, (((1,), (1,)), ((), ())),
                                preferred_element_type=jnp.float32,
                                precision=_HI)
        alpha_ref[pl.ds(i, 1), :] = a_row

    @pl.when(i == MID)
    def _mid_phase():
        v = v_ref[...]
        vsn_out[0:1, :] = v

        s = s_ref[0, 0]
        ge = (s >= 0.5).astype(jnp.float32)
        xa = v * ge
        xb = v * (1.0 - ge)
        gi = (lax.dot_general(xa, wihT_ref[0:TS, :], (((1,), (0,)), ((), ())),
                              preferred_element_type=jnp.float32, precision=_HI)
              + lax.dot_general(xb, wihT_ref[TS:2 * TS, :],
                                (((1,), (0,)), ((), ())),
                                preferred_element_type=jnp.float32,
                                precision=_HI)
              + s * wihT_ref[2 * TS:2 * TS + 1, :]
              + bih_ref[...])
        hrow = h_ref[...]
        gh = (lax.dot_general(hrow, whhT_ref[...], (((1,), (0,)), ((), ())),
                              preferred_element_type=jnp.float32, precision=_HI)
              + bhh_ref[...])
        r = jax.nn.sigmoid(gi[:, 0:HS] + gh[:, 0:HS])
        z = jax.nn.sigmoid(gi[:, HS:2 * HS] + gh[:, HS:2 * HS])
        n = jnp.tanh(gi[:, 2 * HS:3 * HS] + r * gh[:, 2 * HS:3 * HS])
        h_new = (1.0 - z) * n + z * hrow
        hnew_out[...] = h_new
        hrow_ref[...] = h_new
        pltpu.make_async_copy(hrow_ref, hsn_out.at[pl.ds(T, 1), 0],
                              last_sem).start()

        al = alpha_ref[...]
        m = jnp.max(al)
        key = _sortable(al)

        def bis(_, lohi):
            lo, hi = lohi
            mid = (lo >> 1) + (hi >> 1) + (lo & hi & 1)
            cnt = jnp.sum((key >= mid).astype(jnp.int32))
            return jnp.where(cnt >= K, mid, lo), jnp.where(cnt >= K, hi, mid - 1)

        lo, hi = lax.fori_loop(0, 34, bis, (jnp.min(key), jnp.max(key)))
        w = jnp.where(key >= lo, jnp.exp(al - m), 0.0)
        alpha_ref[...] = w
        stat_ref[0] = jnp.sum(w)
        acc_ref[...] = jnp.zeros((1, HS), jnp.float32)

    @pl.when(i > MID)
    def _hs_phase():
        j = i - (MID + 1)
        pltpu.make_async_copy(hs_ref.at[pl.ds(j * BLK, BLK), 0],
                              hsbuf.at[j], in_sems.at[j]).wait()
        blk = hsbuf[j]
        w_row = alpha_ref[pl.ds(j, 1), :]
        acc_ref[...] += lax.dot_general(w_row, blk, (((1,), (0,)), ((), ())),
                                        preferred_element_type=jnp.float32,
                                        precision=_HI)
        pltpu.make_async_copy(hsbuf.at[j], hsn_out.at[pl.ds(j * BLK, BLK), 0],
                              out_sems.at[j]).start()

    @pl.when(i == NSTEPS - 1)
    def _fin():
        attn = acc_ref[...] / stat_ref[0]
        sc = (jnp.sum(v_ref[...] * wsc_ref[:, 0:TS])
              + jnp.sum(attn * wsc_ref[:, TS:TS + HS])
              + jnp.sum(h_ref[...] * wsc_ref[:, TS + HS:TS + 2 * HS])
              + wsc_ref[0, TS + 2 * HS] * float(K)
              + bsc_ref[0, 0])
        sc_out[...] = sc.reshape(1, 1)
        for j in range(NB):
            pltpu.make_async_copy(hsbuf.at[j],
                                  hsn_out.at[pl.ds(j * BLK, BLK), 0],
                                  out_sems.at[j]).wait()
        pltpu.make_async_copy(hrow_ref, hsn_out.at[pl.ds(T, 1), 0],
                              last_sem).wait()


def _tc_main(v_row, score, h_row, vs, hs2, wihT, whhT, bih, bhh, wsc, bsc):
    whole = lambda shape: pl.BlockSpec(shape, lambda i: tuple(0 for _ in shape))
    return pl.pallas_call(
        _tc_body,
        grid=(NSTEPS,),
        in_specs=[
            whole((1, TS)),
            whole((1, 1)),
            whole((1, HS)),
            pl.BlockSpec((BLK, TS), lambda i: (jnp.minimum(i, NB - 1), 0)),
            pl.BlockSpec(memory_space=pl.ANY),
            whole((2 * TS + 1, 3 * HS)),
            whole((HS, 3 * HS)),
            whole((1, 3 * HS)),
            whole((1, 3 * HS)),
            whole((1, TS + 2 * HS + 1)),
            whole((1, 1)),
        ],
        out_specs=[
            whole((1, 1)),
            whole((1, HS)),
            pl.BlockSpec((BLK, TS), lambda i: (jnp.minimum(i, NB), 0)),
            pl.BlockSpec(memory_space=pl.ANY),
        ],
        out_shape=[
            jax.ShapeDtypeStruct((1, 1), jnp.float32),
            jax.ShapeDtypeStruct((1, HS), jnp.float32),
            jax.ShapeDtypeStruct((T + 1, TS), jnp.float32),
            jax.ShapeDtypeStruct((T + 1, 1, HS), jnp.float32),
        ],
        scratch_shapes=[
            pltpu.VMEM((NB, BLK), jnp.float32),
            pltpu.VMEM((1, HS), jnp.float32),
            pltpu.VMEM((1, HS), jnp.float32),
            pltpu.SMEM((2,), jnp.float32),
            pltpu.VMEM((NB, BLK, HS), jnp.float32),
            pltpu.SemaphoreType.DMA((NB,)),
            pltpu.SemaphoreType.DMA((NB,)),
            pltpu.SemaphoreType.DMA,
        ],
    )(v_row, score, h_row, vs, hs2, wihT, whhT, bih, bhh, wsc, bsc)


def kernel(topic, score, time, h, vs, hs, emb, W_ih, W_hh, b_ih, b_hh,
           W_score, b_score):
    del time
    v = _sc_topic_mean(topic.astype(jnp.int32), emb)
    sc, h_new, vs_new, hs_new = _tc_main(
        v.reshape(1, TS), score.reshape(1, 1), h.reshape(1, HS),
        vs, hs,
        W_ih.T, W_hh.T, b_ih.reshape(1, -1), b_hh.reshape(1, -1),
        W_score, b_score.reshape(1, 1))
    return sc, h_new.reshape(1, 1, HS), vs_new, hs_new


# speedup vs baseline: 1.8094x; 1.0143x over previous
import functools

import jax
import jax.numpy as jnp
from jax import lax
from jax.experimental import pallas as pl
from jax.experimental.pallas import tpu as pltpu
from jax.experimental.pallas import tpu_sc as plsc

TS = 128
HS = 128
K = 64
T = 32768
L = 64

BLK = 8192
NB = T // BLK
MID = NB
NSTEPS = 2 * NB + 1

_HI = jax.lax.Precision.HIGHEST


def _sc_topic_mean(topic, emb):
    mesh = plsc.VectorSubcoreMesh(core_axis_name="c", subcore_axis_name="s")

    @functools.partial(
        pl.kernel,
        out_type=jax.ShapeDtypeStruct((TS,), jnp.float32),
        mesh=mesh,
        scratch_types=[
            pltpu.VMEM((L,), jnp.int32),
            pltpu.VMEM((L, TS), jnp.float32),
            pltpu.VMEM((16,), jnp.float32),
            pltpu.SemaphoreType.DMA,
        ],
    )
    def k(topic_hbm, emb_hbm, out_hbm, idx_v, rows_v, acc_v, sem):
        cid = lax.axis_index("c")
        sid = lax.axis_index("s")

        @pl.when(jnp.logical_and(cid == 0, sid < TS // 16))
        def _():
            pltpu.sync_copy(topic_hbm, idx_v)
            pltpu.async_copy(emb_hbm.at[idx_v], rows_v, sem).wait()
            chunk = pl.ds(sid * 16, 16)

            def body(r, acc):
                return acc + rows_v[r, chunk]

            acc = lax.fori_loop(0, L, body, jnp.zeros((16,), jnp.float32))
            acc_v[...] = acc * (1.0 / L)
            pltpu.sync_copy(acc_v, out_hbm.at[chunk])

    return k(topic, emb)


def _sortable(x):
    u = lax.bitcast_convert_type(x, jnp.int32)
    return jnp.where(u >= 0, u, u ^ jnp.int32(0x7FFFFFFF))


def _tc_body(v_ref, s_ref, h_ref, vs_ref, hs_ref, wihT_ref, whhT_ref,
             bih_ref, bhh_ref, wsc_ref, bsc_ref,
             sc_out, hnew_out, vsn_out, hsn_out,
             alpha_ref, acc_ref, hrow_ref, stat_ref, hsbuf,
             in_sems, out_sems, last_sem):
    i = pl.program_id(0)

    @pl.when(i < NB)
    def _vs_phase():
        pltpu.make_async_copy(hs_ref.at[pl.ds(i * BLK, BLK), 0],
                              hsbuf.at[i], in_sems.at[i]).start()
        blk = vs_ref[...]
        vsn_out[...] = blk
        a_row = lax.dot_general(v_ref[...], blk, (((1,), (1,)), ((), ())),
                                preferred_element_type=jnp.float32,
                                precision=_HI)
        alpha_ref[pl.ds(i, 1), :] = a_row

    @pl.when(i == MID)
    def _mid_phase():
        v = v_ref[...]
        vsn_out[0:1, :] = v

        s = s_ref[0, 0]
        ge = (s >= 0.5).astype(jnp.float32)
        xa = v * ge
        xb = v * (1.0 - ge)
        gi = (lax.dot_general(xa, wihT_ref[0:TS, :], (((1,), (0,)), ((), ())),
                              preferred_element_type=jnp.float32, precision=_HI)
              + lax.dot_general(xb, wihT_ref[TS:2 * TS, :],
                                (((1,), (0,)), ((), ())),
                                preferred_element_type=jnp.float32,
                                precision=_HI)
              + s * wihT_ref[2 * TS:2 * TS + 1, :]
              + bih_ref[...])
        hrow = h_ref[...]
        gh = (lax.dot_general(hrow, whhT_ref[...], (((1,), (0,)), ((), ())),
                              preferred_element_type=jnp.float32, precision=_HI)
              + bhh_ref[...])
        r = jax.nn.sigmoid(gi[:, 0:HS] + gh[:, 0:HS])
        z = jax.nn.sigmoid(gi[:, HS:2 * HS] + gh[:, HS:2 * HS])
        n = jnp.tanh(gi[:, 2 * HS:3 * HS] + r * gh[:, 2 * HS:3 * HS])
        h_new = (1.0 - z) * n + z * hrow
        hnew_out[...] = h_new
        hrow_ref[...] = h_new
        pltpu.make_async_copy(hrow_ref, hsn_out.at[pl.ds(T, 1), 0],
                              last_sem).start()

        al = alpha_ref[...]
        m = jnp.max(al)
        key = _sortable(al)

        def bis(_, lohi):
            lo, hi = lohi
            mid = (lo >> 1) + (hi >> 1) + (lo & hi & 1)
            cnt = jnp.sum((key >= mid).astype(jnp.int32))
            return jnp.where(cnt >= K, mid, lo), jnp.where(cnt >= K, hi, mid - 1)

        lo, hi = lax.fori_loop(0, 34, bis, (jnp.min(key), jnp.max(key)))
        w = jnp.where(key >= lo, jnp.exp(al - m), 0.0)
        alpha_ref[...] = w
        stat_ref[0] = jnp.sum(w)
        acc_ref[...] = jnp.zeros((1, HS), jnp.float32)

    @pl.when(i > MID)
    def _hs_phase():
        j = i - (MID + 1)
        pltpu.make_async_copy(hs_ref.at[pl.ds(j * BLK, BLK), 0],
                              hsbuf.at[j], in_sems.at[j]).wait()
        blk = hsbuf[j]
        w_row = alpha_ref[pl.ds(j, 1), :]
        acc_ref[...] += lax.dot_general(w_row, blk, (((1,), (0,)), ((), ())),
                                        preferred_element_type=jnp.float32,
                                        precision=_HI)
        pltpu.make_async_copy(hsbuf.at[j], hsn_out.at[pl.ds(j * BLK, BLK), 0],
                              out_sems.at[j]).start()

    @pl.when(i == NSTEPS - 1)
    def _fin():
        attn = acc_ref[...] / stat_ref[0]
        sc = (jnp.sum(v_ref[...] * wsc_ref[:, 0:TS])
              + jnp.sum(attn * wsc_ref[:, TS:TS + HS])
              + jnp.sum(h_ref[...] * wsc_ref[:, TS + HS:TS + 2 * HS])
              + wsc_ref[0, TS + 2 * HS] * float(K)
              + bsc_ref[0, 0])
        sc_out[...] = sc.reshape(1, 1)
        for j in range(NB):
            pltpu.make_async_copy(hsbuf.at[j],
                                  hsn_out.at[pl.ds(j * BLK, BLK), 0],
                                  out_sems.at[j]).wait()
        pltpu.make_async_copy(hrow_ref, hsn_out.at[pl.ds(T, 1), 0],
                              last_sem).wait()


def _tc_main(v_row, score, h_row, vs, hs2, wihT, whhT, bih, bhh, wsc, bsc):
    whole = lambda shape: pl.BlockSpec(shape, lambda i: tuple(0 for _ in shape))
    return pl.pallas_call(
        _tc_body,
        grid=(NSTEPS,),
        in_specs=[
            whole((1, TS)),
            whole((1, 1)),
            whole((1, HS)),
            pl.BlockSpec((BLK, TS), lambda i: (jnp.minimum(i, NB - 1), 0)),
            pl.BlockSpec(memory_space=pl.ANY),
            whole((2 * TS + 1, 3 * HS)),
            whole((HS, 3 * HS)),
            whole((1, 3 * HS)),
            whole((1, 3 * HS)),
            whole((1, TS + 2 * HS + 1)),
            whole((1, 1)),
        ],
        out_specs=[
            whole((1, 1)),
            whole((1, HS)),
            pl.BlockSpec((BLK, TS), lambda i: (jnp.minimum(i, NB), 0)),
            pl.BlockSpec(memory_space=pl.ANY),
        ],
        out_shape=[
            jax.ShapeDtypeStruct((1, 1), jnp.float32),
            jax.ShapeDtypeStruct((1, HS), jnp.float32),
            jax.ShapeDtypeStruct((T + 1, TS), jnp.float32),
            jax.ShapeDtypeStruct((T + 1, 1, HS), jnp.float32),
        ],
        scratch_shapes=[
            pltpu.VMEM((NB, BLK), jnp.float32),
            pltpu.VMEM((1, HS), jnp.float32),
            pltpu.VMEM((1, HS), jnp.float32),
            pltpu.SMEM((2,), jnp.float32),
            pltpu.VMEM((NB, BLK, HS), jnp.float32),
            pltpu.SemaphoreType.DMA((NB,)),
            pltpu.SemaphoreType.DMA((NB,)),
            pltpu.SemaphoreType.DMA,
        ],
    )(v_row, score, h_row, vs, hs2, wihT, whhT, bih, bhh, wsc, bsc)


def kernel(topic, score, time, h, vs, hs, emb, W_ih, W_hh, b_ih, b_hh,
           W_score, b_score):
    del time
    v = _sc_topic_mean(topic.astype(jnp.int32), emb)
    sc, h_new, vs_new, hs_new = _tc_main(
        v.reshape(1, TS), score.reshape(1, 1), h.reshape(1, HS),
        vs, hs,
        W_ih.T, W_hh.T, b_ih.reshape(1, -1), b_hh.reshape(1, -1),
        W_score, b_score.reshape(1, 1))
    return sc, h_new.reshape(1, 1, HS), vs_new, hs_new
